# baseline (speedup 1.0000x reference)
import jax
import jax.numpy as jnp
from jax.experimental import pallas as pl
from jax.experimental.pallas import tpu as pltpu


def _mm_kernel(order_ref, cnt_ref, base_ref, eord_ref,
               x_ref, wA_ref, wB_ref, b_ref, o_hbm, oscr, osem):
    G, T, IN_D = x_ref.shape
    HALF = wA_ref.shape[2]
    S = oscr.shape[1]
    E = pl.num_programs(0)
    P = G * S
    k = pl.program_id(0)

    b_row = b_ref[eord_ref[k]]

    def body(j, carry):
        p = order_ref[base_ref[k] + j]
        i = p // S
        s = p % S
        xa = x_ref[i]
        accA = jax.lax.dot_general(
            xa, wA_ref[0, 0],
            dimension_numbers=(((1,), (1,)), ((), ())),
            preferred_element_type=jnp.float32,
        )
        oscr[i, s, :, 0:HALF] = accA + b_row[0:HALF]
        accB = jax.lax.dot_general(
            xa, wB_ref[0, 0],
            dimension_numbers=(((1,), (1,)), ((), ())),
            preferred_element_type=jnp.float32,
        )
        oscr[i, s, :, HALF:2 * HALF] = accB + b_row[HALF:2 * HALF]
        pltpu.make_async_copy(oscr.at[i, s], o_hbm.at[i, s], osem).start()
        return carry

    jax.lax.fori_loop(0, cnt_ref[k], body, 0)

    @pl.when(k == E - 1)
    def _drain():
        for _ in range(P):
            pltpu.make_async_copy(
                oscr.at[0, 0], o_hbm.at[0, 0], osem).wait()


def kernel(x, indices, weight, bias):
    G, S = indices.shape
    E, OUT_D, IN_D = weight.shape
    T = x.shape[-2]
    P = G * S
    HALF = OUT_D // 2

    xr = x.reshape(G, T, IN_D)
    ws = weight.reshape(E, 2, HALF, IN_D)

    flat = indices.reshape(P)
    order = jnp.argsort(flat).astype(jnp.int32)
    onehot = flat[None, :] == jnp.arange(E)[:, None]
    counts = jnp.sum(onehot, axis=1).astype(jnp.int32)
    firstpos = jnp.where(onehot, jnp.arange(P)[None, :], P).min(axis=1)
    eord = jnp.argsort(firstpos).astype(jnp.int32)
    mask = jnp.sort(firstpos) < P
    d = jnp.sum(mask.astype(jnp.int32))
    lastused = jnp.take(eord, d - 1)
    eload = jnp.where(mask, eord, lastused).astype(jnp.int32)
    cntk = jnp.take(counts, eord).astype(jnp.int32)
    base_by_id = jnp.cumsum(counts) - counts
    basek = jnp.take(base_by_id, eord).astype(jnp.int32)

    grid_spec = pltpu.PrefetchScalarGridSpec(
        num_scalar_prefetch=4,
        grid=(E,),
        in_specs=[
            pl.BlockSpec((G, T, IN_D), lambda k, *_: (0, 0, 0)),
            pl.BlockSpec((1, 1, HALF, IN_D),
                         lambda k, order, cnt, base, eord: (eord[k], 0, 0, 0)),
            pl.BlockSpec((1, 1, HALF, IN_D),
                         lambda k, order, cnt, base, eord: (eord[k], 1, 0, 0)),
            pl.BlockSpec((E, OUT_D), lambda k, *_: (0, 0)),
        ],
        out_specs=pl.BlockSpec(memory_space=pl.ANY),
        scratch_shapes=[
            pltpu.VMEM((G, S, T, OUT_D), jnp.float32),
            pltpu.SemaphoreType.DMA,
        ],
    )

    out = pl.pallas_call(
        _mm_kernel,
        grid_spec=grid_spec,
        out_shape=jax.ShapeDtypeStruct((G, S, T, OUT_D), jnp.float32),
    )(order, cntk, basek, eload, xr, ws, ws, bias)

    return out.reshape(1, G, S, T, OUT_D)


# speedup vs baseline: 1.0991x; 1.0991x over previous
import jax
import jax.numpy as jnp
from jax.experimental import pallas as pl
from jax.experimental.pallas import tpu as pltpu


def _mm_kernel(idx_ref, fu_ref, eord_ref, mask_ref,
               x_ref, w_hbm, b_ref, o_ref,
               wscr, wsem):
    E = wscr.shape[0]
    S = idx_ref.shape[1]
    i = pl.program_id(0)

    def _wcopy(e):
        return pltpu.make_async_copy(w_hbm.at[e], wscr.at[e], wsem.at[e])

    @pl.when(i == 0)
    def _issue():
        for k in range(E):
            e = eord_ref[k]

            @pl.when(mask_ref[k] == 1)
            def _start_w():
                _wcopy(e).start()

    xa = x_ref[0]
    for s in range(S):
        e_s = idx_ref[i, s]

        @pl.when(fu_ref[i, s] == 1)
        def _wait_w():
            _wcopy(e_s).wait()

        acc = jax.lax.dot_general(
            xa, wscr[e_s],
            dimension_numbers=(((1,), (1,)), ((), ())),
            preferred_element_type=jnp.float32,
        )
        o_ref[0, s] = acc + b_ref[e_s]


def kernel(x, indices, weight, bias):
    G, S = indices.shape
    E, OUT_D, IN_D = weight.shape
    T = x.shape[-2]
    P = G * S

    xr = x.reshape(G, T, IN_D)

    flat = indices.reshape(P)
    eq = flat[:, None] == flat[None, :]
    first = jnp.argmax(eq, axis=1)
    fu = (first == jnp.arange(P)).astype(jnp.int32).reshape(G, S)
    onehot = flat[None, :] == jnp.arange(E)[:, None]
    firstpos = jnp.where(onehot, jnp.arange(P)[None, :], P).min(axis=1)
    eord = jnp.argsort(firstpos).astype(jnp.int32)
    mask = (jnp.sort(firstpos) < P).astype(jnp.int32)

    grid_spec = pltpu.PrefetchScalarGridSpec(
        num_scalar_prefetch=4,
        grid=(G,),
        in_specs=[
            pl.BlockSpec((1, T, IN_D), lambda i, *_: (i, 0, 0)),
            pl.BlockSpec(memory_space=pl.ANY),
            pl.BlockSpec((E, OUT_D), lambda i, *_: (0, 0)),
        ],
        out_specs=pl.BlockSpec((1, S, T, OUT_D),
                               lambda i, *_: (i, 0, 0, 0)),
        scratch_shapes=[
            pltpu.VMEM((E, OUT_D, IN_D), jnp.float32),
            pltpu.SemaphoreType.DMA((E,)),
        ],
    )

    out = pl.pallas_call(
        _mm_kernel,
        grid_spec=grid_spec,
        out_shape=jax.ShapeDtypeStruct((G, S, T, OUT_D), jnp.float32),
    )(indices, fu, eord, mask, xr, weight, bias)

    return out.reshape(1, G, S, T, OUT_D)


# speedup vs baseline: 1.2208x; 1.1107x over previous
import jax
import jax.numpy as jnp
from jax.experimental import pallas as pl
from jax.experimental.pallas import tpu as pltpu


def _mm_kernel(idx_ref, x_ref, w_hbm, b_ref, o_ref, wscr, wsem):
    E = wscr.shape[0]
    G, S = idx_ref.shape
    P = G * S
    i = pl.program_id(0)

    def _flat(p):
        return idx_ref[p // S, p % S]

    def _wcopy(e):
        return pltpu.make_async_copy(w_hbm.at[e], wscr.at[e], wsem.at[e])

    @pl.when(i == 0)
    def _issue():
        for p in range(P):
            e = _flat(p)
            fu = jnp.bool_(True)
            for q in range(p):
                fu = jnp.logical_and(fu, _flat(q) != e)

            @pl.when(fu)
            def _start_w():
                _wcopy(e).start()

    xa = x_ref[0]
    for s in range(S):
        e_s = idx_ref[i, s]
        p = i * S + s
        fu = jnp.bool_(True)
        for q in range(P):
            clash = jnp.logical_and(q < p, _flat(q) == e_s)
            fu = jnp.logical_and(fu, jnp.logical_not(clash))

        @pl.when(fu)
        def _wait_w():
            _wcopy(e_s).wait()

        acc = jax.lax.dot_general(
            xa, wscr[e_s],
            dimension_numbers=(((1,), (1,)), ((), ())),
            preferred_element_type=jnp.float32,
        )
        o_ref[0, s] = acc + b_ref[e_s]


def kernel(x, indices, weight, bias):
    G, S = indices.shape
    E, OUT_D, IN_D = weight.shape
    T = x.shape[-2]

    xr = x.reshape(G, T, IN_D)

    grid_spec = pltpu.PrefetchScalarGridSpec(
        num_scalar_prefetch=1,
        grid=(G,),
        in_specs=[
            pl.BlockSpec((1, T, IN_D), lambda i, ind: (i, 0, 0)),
            pl.BlockSpec(memory_space=pl.ANY),
            pl.BlockSpec((E, OUT_D), lambda i, ind: (0, 0)),
        ],
        out_specs=pl.BlockSpec((1, S, T, OUT_D),
                               lambda i, ind: (i, 0, 0, 0)),
        scratch_shapes=[
            pltpu.VMEM((E, OUT_D, IN_D), jnp.float32),
            pltpu.SemaphoreType.DMA((E,)),
        ],
    )

    out = pl.pallas_call(
        _mm_kernel,
        grid_spec=grid_spec,
        out_shape=jax.ShapeDtypeStruct((G, S, T, OUT_D), jnp.float32),
    )(indices, xr, weight, bias)

    return out.reshape(1, G, S, T, OUT_D)
